# Initial kernel scaffold; baseline (speedup 1.0000x reference)
import jax
import jax.numpy as jnp
from jax.experimental import pallas as pl


def kernel(input, table):
    raise NotImplementedError("write your pallas kernel here")


# speedup vs baseline: 4.7228x; 4.7228x over previous
import functools

import jax
import jax.numpy as jnp
from jax import lax
from jax.experimental import pallas as pl
from jax.experimental.pallas import tpu as pltpu
from jax.experimental.pallas import tpu_sc as plsc

_CB = 128
_K = 4
_CH = _CB * _K


@functools.lru_cache(maxsize=None)
def _make_gather(N, V, D):
    info = plsc.get_sparse_core_info()
    NC, NS = info.num_cores, info.num_subcores
    NW = NC * NS
    per_w = N // NW
    assert per_w * NW == N
    n_ch = per_w // _CH
    assert n_ch * _CH == per_w
    mesh = plsc.VectorSubcoreMesh(core_axis_name="c", subcore_axis_name="s")

    @functools.partial(
        pl.kernel,
        mesh=mesh,
        compiler_params=pltpu.CompilerParams(use_tc_tiling_on_sc=False),
        out_type=jax.ShapeDtypeStruct((N, D), jnp.float32),
        scratch_types=[
            pltpu.VMEM((_K, _CB), jnp.int32),
            pltpu.VMEM((_CH, D), jnp.float32),
            pltpu.SemaphoreType.DMA,
        ],
    )
    def k(idx_hbm, table_hbm, out_hbm, idx_v, rows_v, sem):
        wid = lax.axis_index("s") * NC + lax.axis_index("c")
        row0 = wid * (per_w // _CB)

        def step(g, carry):
            r = row0 + g * _K
            pltpu.sync_copy(idx_hbm.at[pl.ds(r, _K)], idx_v)
            copies = []
            for j in range(_K):
                copies.append(
                    pltpu.async_copy(
                        table_hbm.at[idx_v.at[j]],
                        rows_v.at[pl.ds(j * _CB, _CB)],
                        sem,
                    )
                )
            for c in copies:
                c.wait()
            pltpu.sync_copy(rows_v, out_hbm.at[pl.ds(r * _CB, _CH)])
            return carry

        lax.fori_loop(0, n_ch, step, 0)

    return k


def kernel(input, table):
    B, H = input.shape
    V, D = table.shape
    N = B * H
    idx2d = input.reshape(N // _CB, _CB).astype(jnp.int32)
    out = _make_gather(N, V, D)(idx2d, table)
    return out.reshape(B, H, D)


# speedup vs baseline: 5.1213x; 1.0844x over previous
import functools

import jax
import jax.numpy as jnp
from jax import lax
from jax.experimental import pallas as pl
from jax.experimental.pallas import tpu as pltpu
from jax.experimental.pallas import tpu_sc as plsc

_CB = 128
_K = 4
_CH = _CB * _K


@functools.lru_cache(maxsize=None)
def _make_gather(N, V, D):
    info = plsc.get_sparse_core_info()
    NC, NS = info.num_cores, info.num_subcores
    NW = NC * NS
    per_w = N // NW
    assert per_w * NW == N
    n_ch = per_w // _CH
    assert n_ch * _CH == per_w and n_ch % 2 == 0
    mesh = plsc.VectorSubcoreMesh(core_axis_name="c", subcore_axis_name="s")

    @functools.partial(
        pl.kernel,
        mesh=mesh,
        compiler_params=pltpu.CompilerParams(use_tc_tiling_on_sc=False),
        out_type=jax.ShapeDtypeStruct((N, D), jnp.float32),
        scratch_types=[
            pltpu.VMEM((2, _K, _CB), jnp.int32),
            pltpu.VMEM((2, _CH, D), jnp.float32),
            pltpu.SemaphoreType.DMA,
            pltpu.SemaphoreType.DMA,
            pltpu.SemaphoreType.DMA,
            pltpu.SemaphoreType.DMA,
            pltpu.SemaphoreType.DMA,
            pltpu.SemaphoreType.DMA,
        ],
    )
    def k(idx_hbm, table_hbm, out_hbm, idx_v, rows_v, sg0, sg1, so0, so1, si0, si1):
        sg = (sg0, sg1)
        so = (so0, so1)
        si = (si0, si1)
        wid = lax.axis_index("s") * NC + lax.axis_index("c")
        row0 = wid * (per_w // _CB)

        def start_idx(g, b):
            pltpu.async_copy(idx_hbm.at[pl.ds(row0 + g * _K, _K)], idx_v.at[b], si[b])

        def wait_idx(b):
            pltpu.make_async_copy(idx_hbm.at[pl.ds(0, _K)], idx_v.at[b], si[b]).wait()

        def start_gathers(b):
            for j in range(_K):
                pltpu.async_copy(
                    table_hbm.at[idx_v.at[b, j]],
                    rows_v.at[b, pl.ds(j * _CB, _CB)],
                    sg[b],
                )

        def wait_gathers(b):
            pltpu.make_async_copy(out_hbm.at[pl.ds(0, _CH)], rows_v.at[b], sg[b]).wait()

        def start_out(g, b):
            pltpu.async_copy(rows_v.at[b], out_hbm.at[pl.ds((row0 + g * _K) * _CB, _CH)], so[b])

        def wait_out(b):
            pltpu.make_async_copy(rows_v.at[b], out_hbm.at[pl.ds(0, _CH)], so[b]).wait()

        def pair(t, prefetch):
            g0 = 2 * t
            wait_gathers(0)
            start_out(g0, 0)
            wait_idx(1)
            start_gathers(1)
            if prefetch:
                start_idx(g0 + 2, 0)
            wait_gathers(1)
            start_out(g0 + 1, 1)
            if prefetch:
                start_idx(g0 + 3, 1)
            wait_out(0)
            if prefetch:
                wait_idx(0)
                start_gathers(0)
            wait_out(1)

        start_idx(0, 0)
        wait_idx(0)
        start_gathers(0)
        start_idx(1, 1)
        lax.fori_loop(0, n_ch // 2 - 1, lambda t, c: (pair(t, True), c)[1], 0)
        pair(n_ch // 2 - 1, False)

    return k


def kernel(input, table):
    B, H = input.shape
    V, D = table.shape
    N = B * H
    idx2d = input.reshape(N // _CB, _CB).astype(jnp.int32)
    out = _make_gather(N, V, D)(idx2d, table)
    return out.reshape(B, H, D)
